# baseline (speedup 1.0000x reference)
import jax
import jax.numpy as jnp
from jax.experimental import pallas as pl

N_NODES = 28
IN_DIM = 512
OUT_DIM = 512
BN = 128


def _gcn_fused(x_ref, s_ref, w_ref, o_ref):
    pre = jnp.dot(x_ref[...], w_ref[...], preferred_element_type=jnp.float32)
    out = jnp.dot(s_ref[...], pre, preferred_element_type=jnp.float32)
    o_ref[...] = jnp.maximum(out, 0.0)


def kernel(inputs, weight):
    x = inputs[:, :IN_DIM]
    support = inputs[:, IN_DIM:]
    return pl.pallas_call(
        _gcn_fused,
        grid=(OUT_DIM // BN,),
        in_specs=[
            pl.BlockSpec((N_NODES, IN_DIM), lambda j: (0, 0)),
            pl.BlockSpec((N_NODES, N_NODES), lambda j: (0, 0)),
            pl.BlockSpec((IN_DIM, BN), lambda j: (0, j)),
        ],
        out_specs=pl.BlockSpec((N_NODES, BN), lambda j: (0, j)),
        out_shape=jax.ShapeDtypeStruct((N_NODES, OUT_DIM), jnp.float32),
    )(x, support, weight)


# speedup vs baseline: 2.9407x; 2.9407x over previous
import jax
import jax.numpy as jnp
from jax.experimental import pallas as pl

N_NODES = 28
IN_DIM = 512
OUT_DIM = 512


def _gcn_fused(inputs_ref, w_ref, o_ref):
    packed = inputs_ref[...]
    x = packed[:, :IN_DIM]
    support = packed[:, IN_DIM:]
    pre = jnp.dot(x, w_ref[...], preferred_element_type=jnp.float32)
    out = jnp.dot(support, pre, preferred_element_type=jnp.float32)
    o_ref[...] = jnp.maximum(out, 0.0)


def kernel(inputs, weight):
    return pl.pallas_call(
        _gcn_fused,
        out_shape=jax.ShapeDtypeStruct((N_NODES, OUT_DIM), jnp.float32),
    )(inputs, weight)
